# baseline (speedup 1.0000x reference)
import jax
import jax.numpy as jnp
import numpy as np
from jax import lax
from jax.experimental import pallas as pl
from jax.experimental.pallas import tpu as pltpu
from jax.experimental.pallas import tpu_sc as plsc

N_NODES = 248
N_EDGES = 4096
_NPAD = 256
_ROWS = N_EDGES // 128
_MBLK = N_NODES * 128
_MOUT = 2 * _MBLK

_W2PAD_NP = np.zeros((180, 8), np.float32)


def _sc_body(edges_hbm, out_hbm,
             edges_v, dst2_v, ones_v, zed_v, deg_v, dinv_v,
             norm_v, fidx_v, m_sh, deg_sh, sem):
    cid = lax.axis_index("c")
    sid = lax.axis_index("s")
    zchunk = _MOUT // 16

    @pl.when(cid == 0)
    def _():
        t = sid

        handles = []
        for slot in range(2):
            j = t + slot * 16
            handles.append(
                pltpu.async_copy(edges_hbm.at[pl.ds(j * 256, 256)],
                                 edges_v.at[pl.ds(slot * 256, 256)], sem))
            handles.append(
                pltpu.async_copy(edges_hbm.at[pl.ds(j * 256 + 128, 128)],
                                 dst2_v.at[slot], sem))

        def _ones_it(i, _):
            ones_v[pl.ds(i * 16, 16)] = jnp.full((16,), 1.0, jnp.float32)
            return 0

        lax.fori_loop(0, _NPAD // 16, _ones_it, 0)

        def _zed_it(i, _):
            zed_v[pl.ds(i * 16, 16)] = jnp.zeros((16,), jnp.float32)
            return 0

        lax.fori_loop(0, zchunk // 16, _zed_it, 0)

        pltpu.sync_copy(zed_v, m_sh.at[pl.ds(t * zchunk, zchunk)])

        @pl.when(t == 0)
        def _():
            pltpu.sync_copy(zed_v.at[pl.ds(0, _NPAD)], deg_sh)
        for h in handles:
            h.wait()
        plsc.subcore_barrier()

        handles = []
        for slot in range(2):
            handles.append(
                pltpu.async_copy(ones_v.at[pl.ds(slot * 128, 128)],
                                 deg_sh.at[dst2_v.at[slot]], sem, add=True))
        for h in handles:
            h.wait()
        plsc.subcore_barrier()

        pltpu.sync_copy(deg_sh, deg_v)

        def _dinv_it(i, _):
            d = deg_v[pl.ds(i * 16, 16)] + 1.0
            di = plsc.bitcast(d, jnp.int32)
            y = plsc.bitcast(0x5F3759DF - (di >> 1), jnp.float32)
            for _ in range(3):
                y = y * (1.5 - 0.5 * d * y * y)
            dinv_v[pl.ds(i * 16, 16)] = y
            return 0

        lax.fori_loop(0, _NPAD // 16, _dinv_it, 0)

        for slot in range(2):
            def _edge_it(k, _, slot=slot):
                s16 = edges_v[pl.ds(slot * 256 + k * 16, 16)]
                d16 = edges_v[pl.ds(slot * 256 + 128 + k * 16, 16)]
                ns = plsc.load_gather(dinv_v, [s16])
                nd = plsc.load_gather(dinv_v, [d16])
                norm_v[slot, pl.ds(k * 16, 16)] = ns * nd
                fidx_v[slot, pl.ds(k * 16, 16)] = (
                    s16 * 128 + d16
                    + jnp.where(d16 >= 128, _MBLK - 128, 0))
                return 0

            lax.fori_loop(0, 8, _edge_it, 0)

        @pl.when(t < 2)
        def _():
            def _self_it(k, _):
                jvec = lax.iota(jnp.int32, 16) + t * 128 + k * 16
                valid = jvec < N_NODES
                dv = dinv_v[pl.ds(t * 128 + k * 16, 16)]
                sidx = jvec * 129 + jnp.where(jvec >= 128, _MBLK - 128, 0)
                norm_v[2, pl.ds(k * 16, 16)] = jnp.where(valid, dv * dv, 0.0)
                fidx_v[2, pl.ds(k * 16, 16)] = jnp.where(
                    valid, sidx, _MOUT - 1)
                return 0

            lax.fori_loop(0, 8, _self_it, 0)

        handles = []
        for slot in range(2):
            handles.append(
                pltpu.async_copy(norm_v.at[slot], m_sh.at[fidx_v.at[slot]],
                                 sem, add=True))

        @pl.when(t < 2)
        def _():
            pltpu.sync_copy(norm_v.at[2], m_sh.at[fidx_v.at[2]], add=True)
        for h in handles:
            h.wait()
        plsc.subcore_barrier()

        @pl.when(t == 0)
        def _():
            pltpu.sync_copy(m_sh, out_hbm)


@jax.jit
def _build_operator(edge_index):
    edges1 = jnp.transpose(edge_index.reshape(2, _ROWS, 128),
                           (1, 0, 2)).reshape(2 * N_EDGES)
    mesh = plsc.VectorSubcoreMesh(core_axis_name="c", subcore_axis_name="s")
    m_flat = pl.kernel(
        _sc_body,
        out_type=jax.ShapeDtypeStruct((_MOUT,), jnp.float32),
        mesh=mesh,
        compiler_params=pltpu.CompilerParams(needs_layout_passes=False,
                                             skip_device_barrier=True),
        scratch_types=[
            pltpu.VMEM((512,), jnp.int32),
            pltpu.VMEM((2, 128), jnp.int32),
            pltpu.VMEM((_NPAD,), jnp.float32),
            pltpu.VMEM((_MOUT // 16,), jnp.float32),
            pltpu.VMEM((_NPAD,), jnp.float32),
            pltpu.VMEM((_NPAD,), jnp.float32),
            pltpu.VMEM((3, 128), jnp.float32),
            pltpu.VMEM((3, 128), jnp.int32),
            pltpu.VMEM_SHARED((_MOUT,), jnp.float32),
            pltpu.VMEM_SHARED((_NPAD,), jnp.float32),
            pltpu.SemaphoreType.DMA,
        ],
    )(edges1)
    return m_flat.reshape(2 * N_NODES, 128)


def _tc_body(params_ref, m_ref, w2_ref, b2_ref, w3_ref, b3_ref, xt_ref,
             out_ref, xp_ref):
    s1 = params_ref[0]
    b1 = params_ref[1]
    s2 = params_ref[2]
    bb = params_ref[3]
    xv = xt_ref[...]
    xp_ref[...] = xv
    mv = m_ref[...]
    ha = lax.dot_general(mv[0:N_NODES, :], xv, (((0,), (0,)), ((), ())),
                         preferred_element_type=jnp.float32)
    hb = lax.dot_general(mv[N_NODES:2 * N_NODES, :], xv,
                         (((0,), (0,)), ((), ())),
                         preferred_element_type=jnp.float32)
    h = jnp.concatenate([ha, hb], axis=0)
    h = jnp.tanh(h * s1 + b1) * s2 + bb
    h = lax.dot_general(w2_ref[...], h, (((1,), (0,)), ((), ())),
                        preferred_element_type=jnp.float32) + b2_ref[...]
    h = jnp.tanh(h)
    out_ref[...] = lax.dot_general(w3_ref[...], h, (((1,), (0,)), ((), ())),
                                   preferred_element_type=jnp.float32) + b3_ref[...]


def _fused_mlp(params, mt, w2, b2, w3, b3, xt, block_b=8192):
    n = xt.shape[0]
    batch = xt.shape[1]
    d2 = w2.shape[0]
    d3 = w3.shape[0]
    grid = (batch // block_b,)
    return pl.pallas_call(
        _tc_body,
        grid=grid,
        in_specs=[
            pl.BlockSpec(memory_space=pltpu.SMEM),
            pl.BlockSpec((2 * n, 128), lambda i: (0, 0)),
            pl.BlockSpec((d2, 2 * 128), lambda i: (0, 0)),
            pl.BlockSpec((d2, 1), lambda i: (0, 0)),
            pl.BlockSpec((d3, d2), lambda i: (0, 0)),
            pl.BlockSpec((d3, 1), lambda i: (0, 0)),
            pl.BlockSpec((n, block_b), lambda i: (0, i)),
        ],
        out_specs=[pl.BlockSpec((d3, block_b), lambda i: (0, i)),
                   pl.BlockSpec((n, block_b), lambda i: (0, i))],
        out_shape=[jax.ShapeDtypeStruct((d3, batch), jnp.float32),
                   jax.ShapeDtypeStruct((n, batch), jnp.float32)],
        compiler_params=pltpu.CompilerParams(
            dimension_semantics=("arbitrary",),
            fuse_transposed_lhs_in_matmul=True),
    )(params, mt, w2, b2, w3, b3, xt)


def kernel(x, edge_index, gcn_w, gcn_b, bn_gamma, bn_beta, w2, b2, w3, b3):
    batch = x.shape[0]
    mt = _build_operator(edge_index)
    params = jnp.stack([
        gcn_w[0, 0],
        gcn_b[0],
        bn_gamma[0] * lax.rsqrt(jnp.float32(1.0 + 1e-5)),
        bn_beta[0],
    ])
    xt = jnp.transpose(x.reshape(batch, N_NODES))
    w2p = jnp.concatenate([w2, _W2PAD_NP], axis=1)
    out_t, xp_t = _fused_mlp(params, mt, w2p, b2.reshape(-1, 1), w3,
                             b3.reshape(-1, 1), xt)
    d3 = w3.shape[0]
    out = jnp.transpose(out_t).reshape(batch, 1, d3)
    xp = jnp.transpose(xp_t).reshape(batch, 1, N_NODES)
    return (xp, out)


# speedup vs baseline: 1.0022x; 1.0022x over previous
import jax
import jax.numpy as jnp
import numpy as np
from jax import lax
from jax.experimental import pallas as pl
from jax.experimental.pallas import tpu as pltpu
from jax.experimental.pallas import tpu_sc as plsc

N_NODES = 248
N_EDGES = 4096
_NPAD = 256
_ROWS = N_EDGES // 128
_MBLK = N_NODES * 128
_MOUT = 2 * _MBLK

_W2PAD_NP = np.zeros((180, 8), np.float32)


def _sc_body(edges_hbm, out_hbm,
             edges_v, dst2_v, ones_v, zed_v, deg_v, dinv_v,
             norm_v, fidx_v, m_sh, deg_sh, sem):
    cid = lax.axis_index("c")
    sid = lax.axis_index("s")
    zchunk = _MOUT // 16

    @pl.when(cid == 0)
    def _():
        t = sid

        handles = []
        for slot in range(2):
            j = t + slot * 16
            handles.append(
                pltpu.async_copy(edges_hbm.at[pl.ds(j * 256, 256)],
                                 edges_v.at[pl.ds(slot * 256, 256)], sem))
            handles.append(
                pltpu.async_copy(edges_hbm.at[pl.ds(j * 256 + 128, 128)],
                                 dst2_v.at[slot], sem))

        def _ones_it(i, _):
            ones_v[pl.ds(i * 16, 16)] = jnp.full((16,), 1.0, jnp.float32)
            return 0

        lax.fori_loop(0, _NPAD // 16, _ones_it, 0)

        def _zed_it(i, _):
            zed_v[pl.ds(i * 16, 16)] = jnp.zeros((16,), jnp.float32)
            return 0

        lax.fori_loop(0, zchunk // 16, _zed_it, 0)

        pltpu.sync_copy(zed_v, m_sh.at[pl.ds(t * zchunk, zchunk)])

        @pl.when(t == 0)
        def _():
            pltpu.sync_copy(zed_v.at[pl.ds(0, _NPAD)], deg_sh)
        for h in handles:
            h.wait()
        plsc.subcore_barrier()

        handles = []
        for slot in range(2):
            handles.append(
                pltpu.async_copy(ones_v.at[pl.ds(slot * 128, 128)],
                                 deg_sh.at[dst2_v.at[slot]], sem, add=True))
        for h in handles:
            h.wait()
        plsc.subcore_barrier()

        pltpu.sync_copy(deg_sh, deg_v)

        def _dinv_it(i, _):
            d = deg_v[pl.ds(i * 16, 16)] + 1.0
            di = plsc.bitcast(d, jnp.int32)
            y = plsc.bitcast(0x5F3759DF - (di >> 1), jnp.float32)
            for _ in range(3):
                y = y * (1.5 - 0.5 * d * y * y)
            dinv_v[pl.ds(i * 16, 16)] = y
            return 0

        lax.fori_loop(0, _NPAD // 16, _dinv_it, 0)

        for slot in range(2):
            def _edge_it(k, _, slot=slot):
                s16 = edges_v[pl.ds(slot * 256 + k * 16, 16)]
                d16 = edges_v[pl.ds(slot * 256 + 128 + k * 16, 16)]
                ns = plsc.load_gather(dinv_v, [s16])
                nd = plsc.load_gather(dinv_v, [d16])
                norm_v[slot, pl.ds(k * 16, 16)] = ns * nd
                fidx_v[slot, pl.ds(k * 16, 16)] = (
                    s16 * 128 + d16
                    + jnp.where(d16 >= 128, _MBLK - 128, 0))
                return 0

            lax.fori_loop(0, 8, _edge_it, 0)

        @pl.when(t < 2)
        def _():
            def _self_it(k, _):
                jvec = lax.iota(jnp.int32, 16) + t * 128 + k * 16
                valid = jvec < N_NODES
                dv = dinv_v[pl.ds(t * 128 + k * 16, 16)]
                sidx = jvec * 129 + jnp.where(jvec >= 128, _MBLK - 128, 0)
                norm_v[2, pl.ds(k * 16, 16)] = jnp.where(valid, dv * dv, 0.0)
                fidx_v[2, pl.ds(k * 16, 16)] = jnp.where(
                    valid, sidx, _MOUT - 1)
                return 0

            lax.fori_loop(0, 8, _self_it, 0)

        handles = []
        for slot in range(2):
            handles.append(
                pltpu.async_copy(norm_v.at[slot], m_sh.at[fidx_v.at[slot]],
                                 sem, add=True))

        @pl.when(t < 2)
        def _():
            pltpu.sync_copy(norm_v.at[2], m_sh.at[fidx_v.at[2]], add=True)
        for h in handles:
            h.wait()
        plsc.subcore_barrier()

        @pl.when(t == 0)
        def _():
            pltpu.sync_copy(m_sh, out_hbm)


@jax.jit
def _build_operator(edge_index):
    edges1 = jnp.transpose(edge_index.reshape(2, _ROWS, 128),
                           (1, 0, 2)).reshape(2 * N_EDGES)
    mesh = plsc.VectorSubcoreMesh(core_axis_name="c", subcore_axis_name="s")
    m_flat = pl.kernel(
        _sc_body,
        out_type=jax.ShapeDtypeStruct((_MOUT,), jnp.float32),
        mesh=mesh,
        compiler_params=pltpu.CompilerParams(needs_layout_passes=False),
        scratch_types=[
            pltpu.VMEM((512,), jnp.int32),
            pltpu.VMEM((2, 128), jnp.int32),
            pltpu.VMEM((_NPAD,), jnp.float32),
            pltpu.VMEM((_MOUT // 16,), jnp.float32),
            pltpu.VMEM((_NPAD,), jnp.float32),
            pltpu.VMEM((_NPAD,), jnp.float32),
            pltpu.VMEM((3, 128), jnp.float32),
            pltpu.VMEM((3, 128), jnp.int32),
            pltpu.VMEM_SHARED((_MOUT,), jnp.float32),
            pltpu.VMEM_SHARED((_NPAD,), jnp.float32),
            pltpu.SemaphoreType.DMA,
        ],
    )(edges1)
    return m_flat.reshape(2 * N_NODES, 128)


def _tc_body(params_ref, m_ref, w2_ref, b2_ref, w3_ref, b3_ref, xt_ref,
             out_ref, xp_ref):
    s1 = params_ref[0]
    b1 = params_ref[1]
    s2 = params_ref[2]
    bb = params_ref[3]
    xv = xt_ref[...]
    xp_ref[...] = xv
    mv = m_ref[...]
    ha = lax.dot_general(mv[0:N_NODES, :], xv, (((0,), (0,)), ((), ())),
                         preferred_element_type=jnp.float32)
    hb = lax.dot_general(mv[N_NODES:2 * N_NODES, :], xv,
                         (((0,), (0,)), ((), ())),
                         preferred_element_type=jnp.float32)
    h = jnp.concatenate([ha, hb], axis=0)
    h = jnp.tanh(h * s1 + b1) * s2 + bb
    h = lax.dot_general(w2_ref[...], h, (((1,), (0,)), ((), ())),
                        preferred_element_type=jnp.float32) + b2_ref[...]
    h = jnp.tanh(h)
    out_ref[...] = lax.dot_general(w3_ref[...], h, (((1,), (0,)), ((), ())),
                                   preferred_element_type=jnp.float32) + b3_ref[...]


def _fused_mlp(params, mt, w2, b2, w3, b3, xt, block_b=8192):
    n = xt.shape[0]
    batch = xt.shape[1]
    d2 = w2.shape[0]
    d3 = w3.shape[0]
    grid = (batch // block_b,)
    return pl.pallas_call(
        _tc_body,
        grid=grid,
        in_specs=[
            pl.BlockSpec(memory_space=pltpu.SMEM),
            pl.BlockSpec((2 * n, 128), lambda i: (0, 0)),
            pl.BlockSpec((d2, 2 * 128), lambda i: (0, 0)),
            pl.BlockSpec((d2, 1), lambda i: (0, 0)),
            pl.BlockSpec((d3, d2), lambda i: (0, 0)),
            pl.BlockSpec((d3, 1), lambda i: (0, 0)),
            pl.BlockSpec((n, block_b), lambda i: (0, i)),
        ],
        out_specs=[pl.BlockSpec((d3, block_b), lambda i: (0, i)),
                   pl.BlockSpec((n, block_b), lambda i: (0, i))],
        out_shape=[jax.ShapeDtypeStruct((d3, batch), jnp.float32),
                   jax.ShapeDtypeStruct((n, batch), jnp.float32)],
        compiler_params=pltpu.CompilerParams(
            dimension_semantics=("parallel",),
            fuse_transposed_lhs_in_matmul=True),
    )(params, mt, w2, b2, w3, b3, xt)


def kernel(x, edge_index, gcn_w, gcn_b, bn_gamma, bn_beta, w2, b2, w3, b3):
    batch = x.shape[0]
    mt = _build_operator(edge_index)
    params = jnp.stack([
        gcn_w[0, 0],
        gcn_b[0],
        bn_gamma[0] * lax.rsqrt(jnp.float32(1.0 + 1e-5)),
        bn_beta[0],
    ])
    xt = jnp.transpose(x.reshape(batch, N_NODES))
    w2p = jnp.concatenate([w2, _W2PAD_NP], axis=1)
    out_t, xp_t = _fused_mlp(params, mt, w2p, b2.reshape(-1, 1), w3,
                             b3.reshape(-1, 1), xt)
    d3 = w3.shape[0]
    out = jnp.transpose(out_t).reshape(batch, 1, d3)
    xp = jnp.transpose(xp_t).reshape(batch, 1, N_NODES)
    return (xp, out)
